# baseline (speedup 1.0000x reference)
import jax
import jax.numpy as jnp
from jax.experimental import pallas as pl

B, C = 1024, 7
CC = C * C
Q = B * CC
BI = 32


def _body(xrow_ref, wrow_ref, xrep_ref, pat_ref, out_ref):
    xr = xrow_ref[...]
    w = wrow_ref[...]
    p0 = jnp.abs(xr) * w
    d = (jnp.abs(xr - 1.0) - jnp.abs(xr)) * w
    xi = xrep_ref[...]
    idx = (xi * (xi >= 0).astype(xi.dtype)).astype(jnp.int32)
    lio = jax.lax.broadcasted_iota(jnp.int32, (BI, CC), 1)
    oh = (idx == lio % C).astype(jnp.bfloat16)
    m = jax.lax.dot_general(
        oh, pat_ref[...],
        dimension_numbers=(((1,), (0,)), ((), ())),
        preferred_element_type=jnp.float32,
    )
    out_ref[...] = (p0 + m * d).reshape(BI * CC, 8, 128)


def kernel(input, target, code_weights):
    x = input.reshape(B, C)
    xrow = jnp.tile(x, (1, C)).reshape(1, Q)
    wrow = jnp.tile(code_weights, C * B).reshape(1, Q)
    xrep = jnp.repeat(x, C, axis=1)
    q = jnp.arange(Q, dtype=jnp.int32)
    pat = (q[None, :] % CC == jnp.arange(CC, dtype=jnp.int32)[:, None]
           ).astype(jnp.bfloat16)

    out = pl.pallas_call(
        _body,
        grid=(B // BI,),
        in_specs=[
            pl.BlockSpec((1, Q), lambda i: (0, 0)),
            pl.BlockSpec((1, Q), lambda i: (0, 0)),
            pl.BlockSpec((BI, CC), lambda i: (i, 0)),
            pl.BlockSpec((CC, Q), lambda i: (0, 0)),
        ],
        out_specs=pl.BlockSpec((BI * CC, 8, 128), lambda i: (i, 0, 0)),
        out_shape=jax.ShapeDtypeStruct((B * CC, 8, 128), jnp.float32),
    )(xrow, wrow, xrep, pat)
    return out.reshape(B, B, C, C)


# speedup vs baseline: 44.6649x; 44.6649x over previous
import jax
import jax.numpy as jnp
from jax.experimental import pallas as pl
from jax.experimental.pallas import tpu as pltpu

B, C = 1024, 7
CC = C * C


def _body(w_ref, xT_ref, xrep_ref, out_ref, oh_ref):
    c = pl.program_id(0)
    k = pl.program_id(1)

    @pl.when((c == 0) & (k == 0))
    def _():
        xi = xrep_ref[...]
        idx = (xi * (xi >= 0).astype(xi.dtype)).astype(jnp.int32)
        lio = jax.lax.broadcasted_iota(jnp.int32, (B, CC), 1)
        oh_ref[...] = (idx == lio % C).astype(jnp.bfloat16)

    ck = c * C + k
    sel = (jax.lax.broadcasted_iota(jnp.int32, (CC, B), 0) == ck
           ).astype(jnp.bfloat16)
    m = jax.lax.dot_general(
        oh_ref[...], sel,
        dimension_numbers=(((1,), (0,)), ((), ())),
        preferred_element_type=jnp.float32,
    )
    xk = xT_ref[...].reshape(1, B)
    wk = w_ref[k]
    out_ref[...] = (jnp.abs(xk - m) * wk).reshape(1, 1, B, B)


def kernel(input, target, code_weights):
    x = input.reshape(B, C)
    xT = x.T.reshape(C, 1, B)
    xrep = jnp.repeat(x, C, axis=1)

    out = pl.pallas_call(
        _body,
        grid=(C, C),
        in_specs=[
            pl.BlockSpec(memory_space=pltpu.SMEM),
            pl.BlockSpec((1, 1, B), lambda c, k: (k, 0, 0)),
            pl.BlockSpec((B, CC), lambda c, k: (0, 0)),
        ],
        out_specs=pl.BlockSpec((1, 1, B, B), lambda c, k: (c, k, 0, 0)),
        out_shape=jax.ShapeDtypeStruct((C, C, B, B), jnp.float32),
        scratch_shapes=[pltpu.VMEM((B, CC), jnp.bfloat16)],
    )(code_weights, xT, xrep)
    return out.transpose(2, 3, 0, 1)
